# Initial kernel scaffold; baseline (speedup 1.0000x reference)
import jax
import jax.numpy as jnp
from jax.experimental import pallas as pl


def kernel(word_ids, age_ids, gender_ids, ethni_ids, ins_ids, seg_ids, posi_ids, word_w, seg_w, age_w, gender_w, ethni_w, ins_w, posi_w, ln_gamma, ln_beta):
    raise NotImplementedError("write your pallas kernel here")


# speedup vs baseline: 7.7981x; 7.7981x over previous
import functools

import jax
import jax.numpy as jnp
from jax import lax
from jax.experimental import pallas as pl
from jax.experimental.pallas import tpu as pltpu
from jax.experimental.pallas import tpu_sc as plsc

V = 100000
H = 128
SEG = 2
AGE = 120
GEN = 3
ETH = 10
INS = 5
P = 512
B = 1024
S = 200
EPS = 1e-12

NC = 2
NS = 16
L = 16
NW = NC * NS
N = B * S
TPW = N // NW
CH = 128
NCHUNK = TPW // CH
KG = H // L


def _rsqrt(x):
    i = lax.bitcast_convert_type(x, jnp.int32)
    y = lax.bitcast_convert_type(jnp.int32(0x5F3759DF) - (i >> 1), jnp.float32)
    for _ in range(3):
        y = y * (1.5 - 0.5 * x * y * y)
    return y


def _sc_body(word_ids, age_ids, seg_ids, gen_ids, eth_ids, ins_ids, posi_ids,
             word_w, age_w, combo_w, posi_w, gamma, beta,
             out,
             widx, aidx, sidx, gidx, eidx, iidx, pidx, cidx,
             wrows, arows, crows, prows, gb, sem_idx, sem_gat):
    wid = lax.axis_index("s") * NC + lax.axis_index("c")
    base = wid * TPW

    pltpu.sync_copy(gamma, gb.at[0])
    pltpu.sync_copy(beta, gb.at[1])
    gvec = [gb[0, pl.ds(k * L, L)] for k in range(KG)]
    bvec = [gb[1, pl.ds(k * L, L)] for k in range(KG)]

    def chunk_body(c, carry):
        off = base + c * CH
        hs = [pltpu.async_copy(src.at[pl.ds(off, CH)], dst, sem_idx)
              for src, dst in ((word_ids, widx), (age_ids, aidx),
                               (seg_ids, sidx), (gen_ids, gidx),
                               (eth_ids, eidx), (ins_ids, iidx),
                               (posi_ids, pidx))]
        for h in hs:
            h.wait()
        for g in range(CH // L):
            sl = pl.ds(g * L, L)
            cc = ((sidx[sl] * GEN + gidx[sl]) * ETH + eidx[sl]) * INS + iidx[sl]
            cidx[sl] = cc
        gs = [pltpu.async_copy(word_w.at[widx], wrows, sem_gat),
              pltpu.async_copy(age_w.at[aidx], arows, sem_gat),
              pltpu.async_copy(combo_w.at[cidx], crows, sem_gat),
              pltpu.async_copy(posi_w.at[pidx], prows, sem_gat)]
        for h in gs:
            h.wait()

        def token_body(t, tc):
            xs = []
            acc = None
            acc2 = None
            for k in range(KG):
                sl = pl.ds(k * L, L)
                x = (wrows[t, sl] + prows[t, sl]) + (arows[t, sl] + crows[t, sl])
                xs.append(x)
                acc = x if acc is None else acc + x
                acc2 = x * x if acc2 is None else acc2 + x * x
            mean = jnp.sum(acc) * (1.0 / H)
            msq = jnp.sum(acc2) * (1.0 / H)
            mean_v = jnp.full((L,), mean, dtype=jnp.float32)
            var = jnp.maximum(msq - mean * mean, 0.0) + EPS
            rstd_v = _rsqrt(jnp.full((L,), var, dtype=jnp.float32))
            for k in range(KG):
                a_k = rstd_v * gvec[k]
                c_k = bvec[k] - mean_v * a_k
                wrows[t, pl.ds(k * L, L)] = xs[k] * a_k + c_k
            return tc

        lax.fori_loop(0, CH, token_body, 0, unroll=False)
        pltpu.sync_copy(wrows, out.at[pl.ds(off, CH)])
        return carry

    lax.fori_loop(0, NCHUNK, chunk_body, 0, unroll=False)


@jax.jit
def _run(word_ids, age_ids, seg_ids, gen_ids, eth_ids, ins_ids, posi_ids,
         word_w, age_w, combo_w, posi_w, gamma, beta):
    mesh = plsc.VectorSubcoreMesh(core_axis_name="c", subcore_axis_name="s")
    f = pl.kernel(
        _sc_body,
        out_type=jax.ShapeDtypeStruct((N, H), jnp.float32),
        mesh=mesh,
        compiler_params=pltpu.CompilerParams(needs_layout_passes=False),
        scratch_types=[
            pltpu.VMEM((CH,), jnp.int32),
            pltpu.VMEM((CH,), jnp.int32),
            pltpu.VMEM((CH,), jnp.int32),
            pltpu.VMEM((CH,), jnp.int32),
            pltpu.VMEM((CH,), jnp.int32),
            pltpu.VMEM((CH,), jnp.int32),
            pltpu.VMEM((CH,), jnp.int32),
            pltpu.VMEM((CH,), jnp.int32),
            pltpu.VMEM((CH, H), jnp.float32),
            pltpu.VMEM((CH, H), jnp.float32),
            pltpu.VMEM((CH, H), jnp.float32),
            pltpu.VMEM((CH, H), jnp.float32),
            pltpu.VMEM((2, H), jnp.float32),
            pltpu.SemaphoreType.DMA,
            pltpu.SemaphoreType.DMA,
        ],
    )
    return f(word_ids, age_ids, seg_ids, gen_ids, eth_ids, ins_ids, posi_ids,
             word_w, age_w, combo_w, posi_w, gamma, beta)


def kernel(word_ids, age_ids, gender_ids, ethni_ids, ins_ids, seg_ids, posi_ids,
           word_w, seg_w, age_w, gender_w, ethni_w, ins_w, posi_w, ln_gamma, ln_beta):
    combo_w = (seg_w[:, None, None, None, :] + gender_w[None, :, None, None, :]
               + ethni_w[None, None, :, None, :] + ins_w[None, None, None, :, :]
               ).reshape(SEG * GEN * ETH * INS, H)
    flat = lambda x: x.reshape(N).astype(jnp.int32)
    out = _run(flat(word_ids), flat(age_ids), flat(seg_ids), flat(gender_ids),
               flat(ethni_ids), flat(ins_ids), flat(posi_ids),
               word_w, age_w, combo_w, posi_w, ln_gamma, ln_beta)
    return out.reshape(B, S, H)


# speedup vs baseline: 13.1952x; 1.6921x over previous
import jax
import jax.numpy as jnp
from jax import lax
from jax.experimental import pallas as pl
from jax.experimental.pallas import tpu as pltpu
from jax.experimental.pallas import tpu_sc as plsc

V = 100000
H = 128
SEG = 2
AGE = 120
GEN = 3
ETH = 10
INS = 5
P = 512
B = 1024
S = 200
EPS = 1e-12

NC = 2
NS = 16
L = 16
NW = NC * NS
N = B * S
TPW = N // NW
CH = 128
NCHUNK = TPW // CH
KG = H // L
NCAT = SEG * GEN * ETH * INS


def _rsqrt(x):
    i = lax.bitcast_convert_type(x, jnp.int32)
    y = lax.bitcast_convert_type(jnp.int32(0x5F3759DF) - (i >> 1), jnp.float32)
    for _ in range(2):
        y = y * (1.5 - 0.5 * x * y * y)
    return y


def _sc_body(word_ids, age_ids, seg_ids, gen_ids, eth_ids, ins_ids, posi_ids,
             word_w, ac_w, posi_w, gamma, beta,
             out,
             widx, aidx, sidx, gidx, eidx, iidx, pidx, acidx,
             wrows, acrows, prows, gb, sem_idx, sem_gat, sem_out):
    wid = lax.axis_index("s") * NC + lax.axis_index("c")
    base = wid * TPW

    pltpu.sync_copy(gamma, gb.at[0])
    pltpu.sync_copy(beta, gb.at[1])
    gvec = [gb[0, pl.ds(k * L, L)] for k in range(KG)]
    bvec = [gb[1, pl.ds(k * L, L)] for k in range(KG)]

    idx_pairs = ((word_ids, widx), (age_ids, aidx), (seg_ids, sidx),
                 (gen_ids, gidx), (eth_ids, eidx), (ins_ids, iidx),
                 (posi_ids, pidx))

    def stage_idx(g, b):
        off = base + g * CH
        for src, dst in idx_pairs:
            pltpu.async_copy(src.at[pl.ds(off, CH)], dst.at[b], sem_idx)

    def wait_idx(b):
        for src, dst in idx_pairs:
            pltpu.make_async_copy(src.at[pl.ds(0, CH)], dst.at[b], sem_idx).wait()

    def fuse_cats(b):
        for g in range(CH // L):
            sl = pl.ds(g * L, L)
            cc = ((sidx[b, sl] * GEN + gidx[b, sl]) * ETH + eidx[b, sl]) * INS \
                + iidx[b, sl]
            acidx[b, sl] = aidx[b, sl] * NCAT + cc

    def issue_gathers(b):
        pltpu.async_copy(word_w.at[widx.at[b]], wrows.at[b], sem_gat)
        pltpu.async_copy(ac_w.at[acidx.at[b]], acrows.at[b], sem_gat)
        pltpu.async_copy(posi_w.at[pidx.at[b]], prows.at[b], sem_gat)

    def wait_gathers(b):
        pltpu.make_async_copy(word_w.at[pl.ds(0, CH)], wrows.at[b], sem_gat).wait()
        pltpu.make_async_copy(ac_w.at[pl.ds(0, CH)], acrows.at[b], sem_gat).wait()
        pltpu.make_async_copy(posi_w.at[pl.ds(0, CH)], prows.at[b], sem_gat).wait()

    def wait_scatter(b):
        pltpu.make_async_copy(wrows.at[b], out.at[pl.ds(0, CH)], sem_out).wait()

    def compute_chunk(b):
        def token_body(t, tc):
            xs = []
            acc = None
            acc2 = None
            for k in range(KG):
                sl = pl.ds(k * L, L)
                x = (wrows[b, t, sl] + acrows[b, t, sl]) + prows[b, t, sl]
                xs.append(x)
                acc = x if acc is None else acc + x
                acc2 = x * x if acc2 is None else acc2 + x * x
            mean = jnp.sum(acc) * (1.0 / H)
            msq = jnp.sum(acc2) * (1.0 / H)
            mean_v = jnp.full((L,), mean, dtype=jnp.float32)
            var = jnp.maximum(msq - mean * mean, 0.0) + EPS
            rstd_v = _rsqrt(jnp.full((L,), var, dtype=jnp.float32))
            for k in range(KG):
                a_k = rstd_v * gvec[k]
                c_k = bvec[k] - mean_v * a_k
                wrows[b, t, pl.ds(k * L, L)] = xs[k] * a_k + c_k
            return tc

        lax.fori_loop(0, CH, token_body, 0, unroll=2)

    stage_idx(0, 0)
    wait_idx(0)
    fuse_cats(0)
    issue_gathers(0)
    stage_idx(1, 1)

    def pair_body(g2, carry):
        for bsel in range(2):
            g = g2 * 2 + bsel
            nb = 1 - bsel

            @pl.when(g + 1 < NCHUNK)
            def _prefetch():
                wait_idx(nb)
                fuse_cats(nb)

                @pl.when(g >= 1)
                def _():
                    wait_scatter(nb)

                issue_gathers(nb)

            wait_gathers(bsel)

            @pl.when(g + 2 < NCHUNK)
            def _():
                stage_idx(g + 2, bsel)

            compute_chunk(bsel)
            pltpu.async_copy(wrows.at[bsel], out.at[pl.ds(base + g * CH, CH)],
                             sem_out)
        return carry

    lax.fori_loop(0, NCHUNK // 2, pair_body, 0, unroll=False)
    wait_scatter(1)


@jax.jit
def _run(word_ids, age_ids, seg_ids, gen_ids, eth_ids, ins_ids, posi_ids,
         word_w, ac_w, posi_w, gamma, beta):
    mesh = plsc.VectorSubcoreMesh(core_axis_name="c", subcore_axis_name="s")
    f = pl.kernel(
        _sc_body,
        out_type=jax.ShapeDtypeStruct((N, H), jnp.float32),
        mesh=mesh,
        compiler_params=pltpu.CompilerParams(needs_layout_passes=False),
        scratch_types=[
            pltpu.VMEM((2, CH), jnp.int32),
            pltpu.VMEM((2, CH), jnp.int32),
            pltpu.VMEM((2, CH), jnp.int32),
            pltpu.VMEM((2, CH), jnp.int32),
            pltpu.VMEM((2, CH), jnp.int32),
            pltpu.VMEM((2, CH), jnp.int32),
            pltpu.VMEM((2, CH), jnp.int32),
            pltpu.VMEM((2, CH), jnp.int32),
            pltpu.VMEM((2, CH, H), jnp.float32),
            pltpu.VMEM((2, CH, H), jnp.float32),
            pltpu.VMEM((2, CH, H), jnp.float32),
            pltpu.VMEM((2, H), jnp.float32),
            pltpu.SemaphoreType.DMA,
            pltpu.SemaphoreType.DMA,
            pltpu.SemaphoreType.DMA,
        ],
    )
    return f(word_ids, age_ids, seg_ids, gen_ids, eth_ids, ins_ids, posi_ids,
             word_w, ac_w, posi_w, gamma, beta)


def kernel(word_ids, age_ids, gender_ids, ethni_ids, ins_ids, seg_ids, posi_ids,
           word_w, seg_w, age_w, gender_w, ethni_w, ins_w, posi_w, ln_gamma, ln_beta):
    cat_w = (seg_w[:, None, None, None, :] + gender_w[None, :, None, None, :]
             + ethni_w[None, None, :, None, :] + ins_w[None, None, None, :, :]
             ).reshape(NCAT, H)
    ac_w = (age_w[:, None, :] + cat_w[None, :, :]).reshape(AGE * NCAT, H)
    flat = lambda x: x.reshape(N).astype(jnp.int32)
    out = _run(flat(word_ids), flat(age_ids), flat(seg_ids), flat(gender_ids),
               flat(ethni_ids), flat(ins_ids), flat(posi_ids),
               word_w, ac_w, posi_w, ln_gamma, ln_beta)
    return out.reshape(B, S, H)


# speedup vs baseline: 15.3265x; 1.1615x over previous
import jax
import jax.numpy as jnp
from jax import lax
from jax.experimental import pallas as pl
from jax.experimental.pallas import tpu as pltpu
from jax.experimental.pallas import tpu_sc as plsc

V = 100000
H = 128
SEG = 2
AGE = 120
GEN = 3
ETH = 10
INS = 5
P = 512
B = 1024
S = 200
EPS = 1e-12

NC = 2
NS = 16
L = 16
NW = NC * NS
N = B * S
TPW = N // NW
CH = 128
NCHUNK = TPW // CH
KG = H // L
NCAT = SEG * GEN * ETH * INS


def _rsqrt(x):
    i = lax.bitcast_convert_type(x, jnp.int32)
    y = lax.bitcast_convert_type(jnp.int32(0x5F3759DF) - (i >> 1), jnp.float32)
    for _ in range(2):
        y = y * (1.5 - 0.5 * x * y * y)
    return y


def _sc_body(word_ids, age_ids, seg_ids, gen_ids, eth_ids, ins_ids, posi_ids,
             word_w, ac_w, posi_w, gamma, beta,
             out,
             widx, aidx, sidx, gidx, eidx, iidx, pidx, acidx,
             wrows, acrows, prows, gb, statsf, sem_idx, sem_gat, sem_out):
    wid = lax.axis_index("s") * NC + lax.axis_index("c")
    base = wid * TPW

    pltpu.sync_copy(gamma, gb.at[0])
    pltpu.sync_copy(beta, gb.at[1])
    gvec = [gb[0, pl.ds(k * L, L)] for k in range(KG)]
    bvec = [gb[1, pl.ds(k * L, L)] for k in range(KG)]

    idx_pairs = ((word_ids, widx), (age_ids, aidx), (seg_ids, sidx),
                 (gen_ids, gidx), (eth_ids, eidx), (ins_ids, iidx),
                 (posi_ids, pidx))

    def stage_idx(g, b):
        off = base + g * CH
        for src, dst in idx_pairs:
            pltpu.async_copy(src.at[pl.ds(off, CH)], dst.at[b], sem_idx)

    def wait_idx(b):
        for src, dst in idx_pairs:
            pltpu.make_async_copy(src.at[pl.ds(0, CH)], dst.at[b], sem_idx).wait()

    def fuse_cats(b):
        for g in range(CH // L):
            sl = pl.ds(g * L, L)
            cc = ((sidx[b, sl] * GEN + gidx[b, sl]) * ETH + eidx[b, sl]) * INS \
                + iidx[b, sl]
            acidx[b, sl] = aidx[b, sl] * NCAT + cc

    def issue_gathers(b):
        pltpu.async_copy(word_w.at[widx.at[b]], wrows.at[b], sem_gat)
        pltpu.async_copy(ac_w.at[acidx.at[b]], acrows.at[b], sem_gat)
        pltpu.async_copy(posi_w.at[pidx.at[b]], prows.at[b], sem_gat)

    def wait_gathers(b):
        pltpu.make_async_copy(word_w.at[pl.ds(0, CH)], wrows.at[b], sem_gat).wait()
        pltpu.make_async_copy(ac_w.at[pl.ds(0, CH)], acrows.at[b], sem_gat).wait()
        pltpu.make_async_copy(posi_w.at[pl.ds(0, CH)], prows.at[b], sem_gat).wait()

    def wait_scatter(b):
        pltpu.make_async_copy(wrows.at[b], out.at[pl.ds(0, CH)], sem_out).wait()

    def compute_chunk(b):
        iota_ch = lax.iota(jnp.int32, L) * CH

        @plsc.parallel_loop(0, CH, unroll=2)
        def token_sum(t):
            acc = None
            acc2 = None
            for k in range(KG):
                sl = pl.ds(k * L, L)
                x = (wrows[b, t, sl] + acrows[b, t, sl]) + prows[b, t, sl]
                wrows[b, t, sl] = x
                acc = x if acc is None else acc + x
                acc2 = x * x if acc2 is None else acc2 + x * x
            idx_a = iota_ch + jnp.full((L,), t, dtype=jnp.int32)
            plsc.store_scatter(statsf, [idx_a], acc)
            plsc.store_scatter(statsf, [idx_a + (L * CH)], acc2)

        def group_body(g, gc):
            s = None
            s2 = None
            for lane in range(L):
                va = statsf[pl.ds(lane * CH + g * L, L)]
                s = va if s is None else s + va
            for lane in range(L):
                vb = statsf[pl.ds((L + lane) * CH + g * L, L)]
                s2 = vb if s2 is None else s2 + vb
            mean_vec = s * (1.0 / H)
            msq_vec = s2 * (1.0 / H)
            var_vec = jnp.maximum(msq_vec - mean_vec * mean_vec, 0.0) + EPS
            rstd_vec = _rsqrt(var_vec)
            for j in range(L):
                t = g * L + j
                mean_v = jnp.full((L,), mean_vec[j], dtype=jnp.float32)
                rstd_v = jnp.full((L,), rstd_vec[j], dtype=jnp.float32)
                for k in range(KG):
                    sl = pl.ds(k * L, L)
                    xh = (wrows[b, t, sl] - mean_v) * rstd_v
                    wrows[b, t, sl] = xh * gvec[k] + bvec[k]
            return gc

        lax.fori_loop(0, CH // L, group_body, 0)

    stage_idx(0, 0)
    wait_idx(0)
    fuse_cats(0)
    issue_gathers(0)
    stage_idx(1, 1)

    def pair_body(g2, carry):
        for bsel in range(2):
            g = g2 * 2 + bsel
            nb = 1 - bsel

            @pl.when(g + 1 < NCHUNK)
            def _prefetch():
                wait_idx(nb)
                fuse_cats(nb)

                @pl.when(g >= 1)
                def _():
                    wait_scatter(nb)

                issue_gathers(nb)

            wait_gathers(bsel)

            @pl.when(g + 2 < NCHUNK)
            def _():
                stage_idx(g + 2, bsel)

            compute_chunk(bsel)
            pltpu.async_copy(wrows.at[bsel], out.at[pl.ds(base + g * CH, CH)],
                             sem_out)
        return carry

    lax.fori_loop(0, NCHUNK // 2, pair_body, 0, unroll=False)
    wait_scatter(1)


@jax.jit
def _run(word_ids, age_ids, seg_ids, gen_ids, eth_ids, ins_ids, posi_ids,
         word_w, ac_w, posi_w, gamma, beta):
    mesh = plsc.VectorSubcoreMesh(core_axis_name="c", subcore_axis_name="s")
    f = pl.kernel(
        _sc_body,
        out_type=jax.ShapeDtypeStruct((N, H), jnp.float32),
        mesh=mesh,
        compiler_params=pltpu.CompilerParams(needs_layout_passes=False),
        scratch_types=[
            pltpu.VMEM((2, CH), jnp.int32),
            pltpu.VMEM((2, CH), jnp.int32),
            pltpu.VMEM((2, CH), jnp.int32),
            pltpu.VMEM((2, CH), jnp.int32),
            pltpu.VMEM((2, CH), jnp.int32),
            pltpu.VMEM((2, CH), jnp.int32),
            pltpu.VMEM((2, CH), jnp.int32),
            pltpu.VMEM((2, CH), jnp.int32),
            pltpu.VMEM((2, CH, H), jnp.float32),
            pltpu.VMEM((2, CH, H), jnp.float32),
            pltpu.VMEM((2, CH, H), jnp.float32),
            pltpu.VMEM((2, H), jnp.float32),
            pltpu.VMEM((2 * L * CH,), jnp.float32),
            pltpu.SemaphoreType.DMA,
            pltpu.SemaphoreType.DMA,
            pltpu.SemaphoreType.DMA,
        ],
    )
    return f(word_ids, age_ids, seg_ids, gen_ids, eth_ids, ins_ids, posi_ids,
             word_w, ac_w, posi_w, gamma, beta)


def kernel(word_ids, age_ids, gender_ids, ethni_ids, ins_ids, seg_ids, posi_ids,
           word_w, seg_w, age_w, gender_w, ethni_w, ins_w, posi_w, ln_gamma, ln_beta):
    cat_w = (seg_w[:, None, None, None, :] + gender_w[None, :, None, None, :]
             + ethni_w[None, None, :, None, :] + ins_w[None, None, None, :, :]
             ).reshape(NCAT, H)
    ac_w = (age_w[:, None, :] + cat_w[None, :, :]).reshape(AGE * NCAT, H)
    flat = lambda x: x.reshape(N).astype(jnp.int32)
    out = _run(flat(word_ids), flat(age_ids), flat(seg_ids), flat(gender_ids),
               flat(ethni_ids), flat(ins_ids), flat(posi_ids),
               word_w, ac_w, posi_w, ln_gamma, ln_beta)
    return out.reshape(B, S, H)


# speedup vs baseline: 15.6645x; 1.0221x over previous
import jax
import jax.numpy as jnp
from jax import lax
from jax.experimental import pallas as pl
from jax.experimental.pallas import tpu as pltpu
from jax.experimental.pallas import tpu_sc as plsc

V = 100000
H = 128
SEG = 2
AGE = 120
GEN = 3
ETH = 10
INS = 5
P = 512
B = 1024
S = 200
EPS = 1e-12

NC = 2
NS = 16
L = 16
NW = NC * NS
N = B * S
TPW = N // NW
CH = 80
NCHUNK = TPW // CH
KG = H // L
NCAT = SEG * GEN * ETH * INS


def _rsqrt(x):
    i = lax.bitcast_convert_type(x, jnp.int32)
    y = lax.bitcast_convert_type(jnp.int32(0x5F3759DF) - (i >> 1), jnp.float32)
    for _ in range(2):
        y = y * (1.5 - 0.5 * x * y * y)
    return y


def _sc_body(word_ids, age_ids, seg_ids, gen_ids, eth_ids, ins_ids, posi_ids,
             word_w, ac_w, posi_w, gamma, beta,
             out,
             widx, aidx, sidx, gidx, eidx, iidx, pidx, acidx,
             wrows, acrows, posi_v, gb, statsf, sem_idx, sem_gat, sem_out):
    wid = lax.axis_index("s") * NC + lax.axis_index("c")
    base = wid * TPW

    pltpu.sync_copy(gamma, gb.at[0])
    pltpu.sync_copy(beta, gb.at[1])
    pltpu.sync_copy(posi_w, posi_v)
    gvec = [gb[0, pl.ds(k * L, L)] for k in range(KG)]
    bvec = [gb[1, pl.ds(k * L, L)] for k in range(KG)]

    idx_pairs = ((word_ids, widx), (age_ids, aidx), (seg_ids, sidx),
                 (gen_ids, gidx), (eth_ids, eidx), (ins_ids, iidx),
                 (posi_ids, pidx))

    def stage_idx(g, b):
        off = base + g * CH
        for src, dst in idx_pairs:
            pltpu.async_copy(src.at[pl.ds(off, CH)], dst.at[b], sem_idx)

    def wait_idx(b):
        for src, dst in idx_pairs:
            pltpu.make_async_copy(src.at[pl.ds(0, CH)], dst.at[b], sem_idx).wait()

    def fuse_cats(b):
        for g in range(CH // L):
            sl = pl.ds(g * L, L)
            cc = ((sidx[b, sl] * GEN + gidx[b, sl]) * ETH + eidx[b, sl]) * INS \
                + iidx[b, sl]
            acidx[b, sl] = aidx[b, sl] * NCAT + cc

    def issue_gathers(b):
        pltpu.async_copy(word_w.at[widx.at[b]], wrows.at[b], sem_gat)
        pltpu.async_copy(ac_w.at[acidx.at[b]], acrows.at[b], sem_gat)

    def wait_gathers(b):
        pltpu.make_async_copy(word_w.at[pl.ds(0, CH)], wrows.at[b], sem_gat).wait()
        pltpu.make_async_copy(ac_w.at[pl.ds(0, CH)], acrows.at[b], sem_gat).wait()

    def wait_scatter(b):
        pltpu.make_async_copy(wrows.at[b], out.at[pl.ds(0, CH)], sem_out).wait()

    def compute_chunk(b):
        iota_ch = lax.iota(jnp.int32, L) * CH
        iota_l = lax.iota(jnp.int32, L)

        @plsc.parallel_loop(0, CH, unroll=2)
        def token_sum(t):
            tg = jnp.bitwise_and(t, jnp.int32(~(L - 1)))
            lane = jnp.bitwise_and(t, jnp.int32(L - 1))
            pvec = pidx[b, pl.ds(tg, L)]
            pid = jnp.take_along_axis(
                pvec, jnp.full((L,), lane, dtype=jnp.int32), axis=0)
            paddr = pid * H + iota_l
            acc = None
            acc2 = None
            for k in range(KG):
                sl = pl.ds(k * L, L)
                pk = plsc.load_gather(posi_v, [paddr + (k * L)])
                x = (wrows[b, t, sl] + acrows[b, t, sl]) + pk
                wrows[b, t, sl] = x
                acc = x if acc is None else acc + x
                acc2 = x * x if acc2 is None else acc2 + x * x
            idx_a = iota_ch + jnp.full((L,), t, dtype=jnp.int32)
            plsc.store_scatter(statsf, [idx_a], acc)
            plsc.store_scatter(statsf, [idx_a + (L * CH)], acc2)

        def group_body(g, gc):
            s = None
            s2 = None
            for lane in range(L):
                va = statsf[pl.ds(lane * CH + g * L, L)]
                s = va if s is None else s + va
            for lane in range(L):
                vb = statsf[pl.ds((L + lane) * CH + g * L, L)]
                s2 = vb if s2 is None else s2 + vb
            mean_vec = s * (1.0 / H)
            msq_vec = s2 * (1.0 / H)
            var_vec = jnp.maximum(msq_vec - mean_vec * mean_vec, 0.0) + EPS
            rstd_vec = _rsqrt(var_vec)
            for j in range(L):
                t = g * L + j
                mean_v = jnp.full((L,), mean_vec[j], dtype=jnp.float32)
                rstd_v = jnp.full((L,), rstd_vec[j], dtype=jnp.float32)
                for k in range(KG):
                    sl = pl.ds(k * L, L)
                    xh = (wrows[b, t, sl] - mean_v) * rstd_v
                    wrows[b, t, sl] = xh * gvec[k] + bvec[k]
            return gc

        lax.fori_loop(0, CH // L, group_body, 0)

    stage_idx(0, 0)
    wait_idx(0)
    fuse_cats(0)
    issue_gathers(0)
    stage_idx(1, 1)

    def pair_body(g2, carry):
        for bsel in range(2):
            g = g2 * 2 + bsel
            nb = 1 - bsel

            @pl.when(g + 1 < NCHUNK)
            def _prefetch():
                wait_idx(nb)
                fuse_cats(nb)

                @pl.when(g >= 1)
                def _():
                    wait_scatter(nb)

                issue_gathers(nb)

            wait_gathers(bsel)

            @pl.when(g + 2 < NCHUNK)
            def _():
                stage_idx(g + 2, bsel)

            compute_chunk(bsel)
            pltpu.async_copy(wrows.at[bsel], out.at[pl.ds(base + g * CH, CH)],
                             sem_out)
        return carry

    lax.fori_loop(0, NCHUNK // 2, pair_body, 0, unroll=False)
    wait_scatter(1)


@jax.jit
def _run(word_ids, age_ids, seg_ids, gen_ids, eth_ids, ins_ids, posi_ids,
         word_w, ac_w, posi_w, gamma, beta):
    mesh = plsc.VectorSubcoreMesh(core_axis_name="c", subcore_axis_name="s")
    f = pl.kernel(
        _sc_body,
        out_type=jax.ShapeDtypeStruct((N, H), jnp.float32),
        mesh=mesh,
        compiler_params=pltpu.CompilerParams(needs_layout_passes=False),
        scratch_types=[
            pltpu.VMEM((2, CH), jnp.int32),
            pltpu.VMEM((2, CH), jnp.int32),
            pltpu.VMEM((2, CH), jnp.int32),
            pltpu.VMEM((2, CH), jnp.int32),
            pltpu.VMEM((2, CH), jnp.int32),
            pltpu.VMEM((2, CH), jnp.int32),
            pltpu.VMEM((2, CH), jnp.int32),
            pltpu.VMEM((2, CH), jnp.int32),
            pltpu.VMEM((2, CH, H), jnp.float32),
            pltpu.VMEM((2, CH, H), jnp.float32),
            pltpu.VMEM((P * H,), jnp.float32),
            pltpu.VMEM((2, H), jnp.float32),
            pltpu.VMEM((2 * L * CH,), jnp.float32),
            pltpu.SemaphoreType.DMA,
            pltpu.SemaphoreType.DMA,
            pltpu.SemaphoreType.DMA,
        ],
    )
    return f(word_ids, age_ids, seg_ids, gen_ids, eth_ids, ins_ids, posi_ids,
             word_w, ac_w, posi_w, gamma, beta)


def kernel(word_ids, age_ids, gender_ids, ethni_ids, ins_ids, seg_ids, posi_ids,
           word_w, seg_w, age_w, gender_w, ethni_w, ins_w, posi_w, ln_gamma, ln_beta):
    cat_w = (seg_w[:, None, None, None, :] + gender_w[None, :, None, None, :]
             + ethni_w[None, None, :, None, :] + ins_w[None, None, None, :, :]
             ).reshape(NCAT, H)
    ac_w = (age_w[:, None, :] + cat_w[None, :, :]).reshape(AGE * NCAT, H)
    flat = lambda x: x.reshape(N).astype(jnp.int32)
    out = _run(flat(word_ids), flat(age_ids), flat(seg_ids), flat(gender_ids),
               flat(ethni_ids), flat(ins_ids), flat(posi_ids),
               word_w, ac_w, posi_w.reshape(P * H), ln_gamma, ln_beta)
    return out.reshape(B, S, H)
